# Initial kernel scaffold; baseline (speedup 1.0000x reference)
import jax
import jax.numpy as jnp
from jax.experimental import pallas as pl


def kernel(point, optimized_weights, optimized_biases, optimized_function_ops, code):
    raise NotImplementedError("write your pallas kernel here")


# speedup vs baseline: 12.9986x; 12.9986x over previous
import jax
import jax.numpy as jnp
from jax import lax
from jax.experimental import pallas as pl
from jax.experimental.pallas import tpu as pltpu
from jax.experimental.pallas import tpu_sc as plsc

_B = 16384
_P = 200
_NF = 32
_SKIP = 10
_NC = 2
_NS = 16
_NW = _NC * _NS
_L = 16
_CHUNK = _B // _NW
_G = _CHUNK // _L


def _ifs_body(idx_hbm, pt_hbm, tab_hbm, out_hbm, idx_v, tab_v, x_v, y_v,
              stage_v, sem):
    wid = lax.axis_index("s") * _NC + lax.axis_index("c")
    base = wid * _CHUNK

    pltpu.sync_copy(idx_hbm.at[pl.ds(base * _P, _CHUNK * _P)], idx_v)
    pltpu.sync_copy(tab_hbm, tab_v)
    pltpu.sync_copy(pt_hbm.at[0, pl.ds(base, _CHUNK)], x_v)
    pltpu.sync_copy(pt_hbm.at[1, pl.ds(base, _CHUNK)], y_v)

    lanes = lax.iota(jnp.int32, _L)

    def step(p, carry):
        pcol = jnp.full((_L,), p, jnp.int32)

        def group(g, carry):
            rows = g * _L + lanes
            fidx = plsc.load_gather(idx_v, [rows * _P + pcol])
            w00 = plsc.load_gather(tab_v, [fidx])
            w01 = plsc.load_gather(tab_v, [fidx + _NF])
            w10 = plsc.load_gather(tab_v, [fidx + 2 * _NF])
            w11 = plsc.load_gather(tab_v, [fidx + 3 * _NF])
            b0 = plsc.load_gather(tab_v, [fidx + 4 * _NF])
            b1 = plsc.load_gather(tab_v, [fidx + 5 * _NF])
            op = plsc.load_gather(tab_v, [fidx + 6 * _NF])
            x = x_v[pl.ds(g * _L, _L)]
            y = y_v[pl.ds(g * _L, _L)]
            nx = w00 * x + w01 * y + b0
            ny = w10 * x + w11 * y + b1
            x_v[pl.ds(g * _L, _L)] = nx
            y_v[pl.ds(g * _L, _L)] = ny
            pos = rows * 3
            plsc.store_scatter(stage_v, [pos], nx)
            plsc.store_scatter(stage_v, [pos + 1], ny)
            plsc.store_scatter(stage_v, [pos + 2], op)
            return carry

        lax.fori_loop(0, _G, group, 0)

        @pl.when(p >= _SKIP)
        def _():
            off = ((p - _SKIP) * _B + base) * 3
            pltpu.async_copy(stage_v, out_hbm.at[pl.ds(off, _CHUNK * 3)],
                             sem).wait()

        return carry

    lax.fori_loop(0, _P, step, 0)


def kernel(point, optimized_weights, optimized_biases, optimized_function_ops,
           code):
    logits = jnp.log(code + 1e-8)
    index = jax.random.categorical(jax.random.key(123), logits[:, None, :],
                                   axis=-1, shape=(_B, _P)).astype(jnp.int32)

    tab = jnp.concatenate([
        optimized_weights[:, 0, 0], optimized_weights[:, 0, 1],
        optimized_weights[:, 1, 0], optimized_weights[:, 1, 1],
        optimized_biases[:, 0, 0], optimized_biases[:, 1, 0],
        optimized_function_ops,
    ])
    pt_t = point[:, :, 0].T

    mesh = plsc.VectorSubcoreMesh(core_axis_name="c", subcore_axis_name="s")
    run = pl.kernel(
        _ifs_body,
        out_type=jax.ShapeDtypeStruct(((_P - _SKIP) * _B * 3,), jnp.float32),
        mesh=mesh,
        compiler_params=pltpu.CompilerParams(needs_layout_passes=False),
        scratch_types=[
            pltpu.VMEM((_CHUNK * _P,), jnp.int32),
            pltpu.VMEM((224,), jnp.float32),
            pltpu.VMEM((_CHUNK,), jnp.float32),
            pltpu.VMEM((_CHUNK,), jnp.float32),
            pltpu.VMEM((_CHUNK * 3,), jnp.float32),
            pltpu.SemaphoreType.DMA,
        ],
    )
    out = run(index.reshape(-1), pt_t, tab)
    return out.reshape((_P - _SKIP) * _B, 3)


# speedup vs baseline: 13.0034x; 1.0004x over previous
import jax
import jax.numpy as jnp
from jax import lax
from jax.experimental import pallas as pl
from jax.experimental.pallas import tpu as pltpu
from jax.experimental.pallas import tpu_sc as plsc

_B = 16384
_P = 200
_NF = 32
_SKIP = 10
_NC = 2
_NS = 16
_NW = _NC * _NS
_L = 16
_CHUNK = _B // _NW
_G = _CHUNK // _L


def _ifs_body(idx_hbm, pt_hbm, tab_hbm, out_hbm, idx_v, tab_v, x_v, y_v,
              stage_v, sem):
    wid = lax.axis_index("s") * _NC + lax.axis_index("c")
    base = wid * _CHUNK

    pltpu.sync_copy(idx_hbm.at[pl.ds(base, _CHUNK), :], idx_v)
    pltpu.sync_copy(tab_hbm, tab_v)
    pltpu.sync_copy(pt_hbm.at[0, pl.ds(base, _CHUNK)], x_v)
    pltpu.sync_copy(pt_hbm.at[1, pl.ds(base, _CHUNK)], y_v)

    lanes = lax.iota(jnp.int32, _L)

    def step(p, carry):
        pcol = jnp.full((_L,), p, jnp.int32)

        def group(g, carry):
            rows = g * _L + lanes
            fidx = plsc.load_gather(idx_v, [rows, pcol])
            w00 = plsc.load_gather(tab_v, [fidx])
            w01 = plsc.load_gather(tab_v, [fidx + _NF])
            w10 = plsc.load_gather(tab_v, [fidx + 2 * _NF])
            w11 = plsc.load_gather(tab_v, [fidx + 3 * _NF])
            b0 = plsc.load_gather(tab_v, [fidx + 4 * _NF])
            b1 = plsc.load_gather(tab_v, [fidx + 5 * _NF])
            op = plsc.load_gather(tab_v, [fidx + 6 * _NF])
            x = x_v[pl.ds(g * _L, _L)]
            y = y_v[pl.ds(g * _L, _L)]
            nx = w00 * x + w01 * y + b0
            ny = w10 * x + w11 * y + b1
            x_v[pl.ds(g * _L, _L)] = nx
            y_v[pl.ds(g * _L, _L)] = ny
            pos = rows * 3
            plsc.store_scatter(stage_v, [pos], nx)
            plsc.store_scatter(stage_v, [pos + 1], ny)
            plsc.store_scatter(stage_v, [pos + 2], op)
            return carry

        lax.fori_loop(0, _G, group, 0)

        @pl.when(p >= _SKIP)
        def _():
            off = ((p - _SKIP) * _B + base) * 3
            pltpu.async_copy(stage_v, out_hbm.at[pl.ds(off, _CHUNK * 3)],
                             sem).wait()

        return carry

    lax.fori_loop(0, _P, step, 0)


def kernel(point, optimized_weights, optimized_biases, optimized_function_ops,
           code):
    logits = jnp.log(code + 1e-8)
    index = jax.random.categorical(jax.random.key(123), logits[:, None, :],
                                   axis=-1, shape=(_B, _P)).astype(jnp.int32)

    tab = jnp.concatenate([
        optimized_weights[:, 0, 0], optimized_weights[:, 0, 1],
        optimized_weights[:, 1, 0], optimized_weights[:, 1, 1],
        optimized_biases[:, 0, 0], optimized_biases[:, 1, 0],
        optimized_function_ops,
    ])
    pt_t = point[:, :, 0].T

    mesh = plsc.VectorSubcoreMesh(core_axis_name="c", subcore_axis_name="s")
    run = pl.kernel(
        _ifs_body,
        out_type=jax.ShapeDtypeStruct(((_P - _SKIP) * _B * 3,), jnp.float32),
        mesh=mesh,
        compiler_params=pltpu.CompilerParams(needs_layout_passes=False,
                                             use_tc_tiling_on_sc=False),
        scratch_types=[
            pltpu.VMEM((_CHUNK, _P), jnp.int32),
            pltpu.VMEM((224,), jnp.float32),
            pltpu.VMEM((_CHUNK,), jnp.float32),
            pltpu.VMEM((_CHUNK,), jnp.float32),
            pltpu.VMEM((_CHUNK * 3,), jnp.float32),
            pltpu.SemaphoreType.DMA,
        ],
    )
    out = run(index, pt_t, tab)
    return out.reshape((_P - _SKIP) * _B, 3)


# speedup vs baseline: 14.9016x; 1.1460x over previous
import jax
import jax.numpy as jnp
from jax import lax
from jax.experimental import pallas as pl
from jax.experimental.pallas import tpu as pltpu
from jax.experimental.pallas import tpu_sc as plsc

_B = 16384
_P = 200
_NF = 32
_SKIP = 10
_NC = 2
_NS = 16
_NW = _NC * _NS
_L = 16
_CHUNK = _B // _NW
_G = _CHUNK // _L


def _ifs_body(idx_hbm, pt_hbm, tab_hbm, out_hbm, idx_v, tab_v, x_v, y_v,
              stage_v, sem):
    wid = lax.axis_index("s") * _NC + lax.axis_index("c")
    base = wid * _CHUNK

    pltpu.sync_copy(idx_hbm.at[pl.ds(base, _CHUNK), :], idx_v)
    pltpu.sync_copy(tab_hbm, tab_v)
    pltpu.sync_copy(pt_hbm.at[0, pl.ds(base, _CHUNK)], x_v)
    pltpu.sync_copy(pt_hbm.at[1, pl.ds(base, _CHUNK)], y_v)

    lanes = lax.iota(jnp.int32, _L)

    def step(p, carry):
        pcol = jnp.full((_L,), p, jnp.int32)

        def group(g, carry):
            rows = g * _L + lanes
            fidx = plsc.load_gather(idx_v, [rows, pcol])
            w00 = plsc.load_gather(tab_v, [fidx])
            w01 = plsc.load_gather(tab_v, [fidx + _NF])
            w10 = plsc.load_gather(tab_v, [fidx + 2 * _NF])
            w11 = plsc.load_gather(tab_v, [fidx + 3 * _NF])
            b0 = plsc.load_gather(tab_v, [fidx + 4 * _NF])
            b1 = plsc.load_gather(tab_v, [fidx + 5 * _NF])
            op = plsc.load_gather(tab_v, [fidx + 6 * _NF])
            x = x_v[pl.ds(g * _L, _L)]
            y = y_v[pl.ds(g * _L, _L)]
            nx = w00 * x + w01 * y + b0
            ny = w10 * x + w11 * y + b1
            x_v[pl.ds(g * _L, _L)] = nx
            y_v[pl.ds(g * _L, _L)] = ny
            plsc.store_scatter(stage_v, [rows, jnp.zeros((_L,), jnp.int32)], nx)
            plsc.store_scatter(stage_v, [rows, jnp.ones((_L,), jnp.int32)], ny)
            plsc.store_scatter(stage_v, [rows, jnp.full((_L,), 2, jnp.int32)], op)
            return carry

        lax.fori_loop(0, _G, group, 0)

        @pl.when(p >= _SKIP)
        def _():
            off = (p - _SKIP) * _B + base
            pltpu.async_copy(stage_v, out_hbm.at[pl.ds(off, _CHUNK), :],
                             sem).wait()

        return carry

    lax.fori_loop(0, _P, step, 0)


def kernel(point, optimized_weights, optimized_biases, optimized_function_ops,
           code):
    logits = jnp.log(code + 1e-8)
    index = jax.random.categorical(jax.random.key(123), logits[:, None, :],
                                   axis=-1, shape=(_B, _P)).astype(jnp.int32)

    tab = jnp.concatenate([
        optimized_weights[:, 0, 0], optimized_weights[:, 0, 1],
        optimized_weights[:, 1, 0], optimized_weights[:, 1, 1],
        optimized_biases[:, 0, 0], optimized_biases[:, 1, 0],
        optimized_function_ops,
    ])
    pt_t = point[:, :, 0].T

    mesh = plsc.VectorSubcoreMesh(core_axis_name="c", subcore_axis_name="s")
    run = pl.kernel(
        _ifs_body,
        out_type=jax.ShapeDtypeStruct(((_P - _SKIP) * _B, 3), jnp.float32),
        mesh=mesh,
        compiler_params=pltpu.CompilerParams(needs_layout_passes=False,
                                             use_tc_tiling_on_sc=False),
        scratch_types=[
            pltpu.VMEM((_CHUNK, _P), jnp.int32),
            pltpu.VMEM((224,), jnp.float32),
            pltpu.VMEM((_CHUNK,), jnp.float32),
            pltpu.VMEM((_CHUNK,), jnp.float32),
            pltpu.VMEM((_CHUNK, 3), jnp.float32),
            pltpu.SemaphoreType.DMA,
        ],
    )
    return run(index, pt_t, tab)


# speedup vs baseline: 15.3413x; 1.0295x over previous
import jax
import jax.numpy as jnp
from jax import lax
from jax.experimental import pallas as pl
from jax.experimental.pallas import tpu as pltpu
from jax.experimental.pallas import tpu_sc as plsc

_B = 16384
_P = 200
_NF = 32
_SKIP = 10
_NC = 2
_NS = 16
_NW = _NC * _NS
_L = 16
_CHUNK = _B // _NW
_HALF = _CHUNK // 2
_G = _HALF // _L


def _ifs_body(idx_hbm, pt_hbm, tab_hbm, out_hbm, idx_v, tab_v, x_v, y_v,
              stage_v, sem):
    wid = lax.axis_index("s") * _NC + lax.axis_index("c")
    base = wid * _CHUNK

    pltpu.sync_copy(tab_hbm, tab_v)
    lanes = lax.iota(jnp.int32, _L)

    for h in range(2):
        hb = base + h * _HALF
        pltpu.sync_copy(idx_hbm.at[pl.ds(hb, _HALF), :], idx_v)
        pltpu.sync_copy(pt_hbm.at[0, pl.ds(hb, _HALF)], x_v)
        pltpu.sync_copy(pt_hbm.at[1, pl.ds(hb, _HALF)], y_v)

        def step(p, carry):
            pcol = jnp.full((_L,), p, jnp.int32)

            def group(g, carry):
                rows = g * _L + lanes
                fidx = plsc.load_gather(idx_v, [rows, pcol])
                w00 = plsc.load_gather(tab_v, [fidx])
                w01 = plsc.load_gather(tab_v, [fidx + _NF])
                w10 = plsc.load_gather(tab_v, [fidx + 2 * _NF])
                w11 = plsc.load_gather(tab_v, [fidx + 3 * _NF])
                b0 = plsc.load_gather(tab_v, [fidx + 4 * _NF])
                b1 = plsc.load_gather(tab_v, [fidx + 5 * _NF])
                op = plsc.load_gather(tab_v, [fidx + 6 * _NF])
                x = x_v[pl.ds(g * _L, _L)]
                y = y_v[pl.ds(g * _L, _L)]
                nx = w00 * x + w01 * y + b0
                ny = w10 * x + w11 * y + b1
                x_v[pl.ds(g * _L, _L)] = nx
                y_v[pl.ds(g * _L, _L)] = ny
                plsc.store_scatter(stage_v,
                                   [rows, jnp.zeros((_L,), jnp.int32)], nx)
                plsc.store_scatter(stage_v,
                                   [rows, jnp.ones((_L,), jnp.int32)], ny)
                plsc.store_scatter(stage_v,
                                   [rows, jnp.full((_L,), 2, jnp.int32)], op)
                return carry

            lax.fori_loop(0, _G, group, 0)

            @pl.when(p >= _SKIP)
            def _():
                off = (p - _SKIP) * _B + hb
                pltpu.async_copy(stage_v, out_hbm.at[pl.ds(off, _HALF), :],
                                 sem).wait()

            return carry

        lax.fori_loop(0, _P, step, 0)


def kernel(point, optimized_weights, optimized_biases, optimized_function_ops,
           code):
    logits = jnp.log(code + 1e-8)
    index = jax.random.categorical(jax.random.key(123), logits[:, None, :],
                                   axis=-1, shape=(_B, _P)).astype(jnp.int32)

    tab = jnp.concatenate([
        optimized_weights[:, 0, 0], optimized_weights[:, 0, 1],
        optimized_weights[:, 1, 0], optimized_weights[:, 1, 1],
        optimized_biases[:, 0, 0], optimized_biases[:, 1, 0],
        optimized_function_ops,
    ])
    pt_t = point[:, :, 0].T

    mesh = plsc.VectorSubcoreMesh(core_axis_name="c", subcore_axis_name="s")
    run = pl.kernel(
        _ifs_body,
        out_type=jax.ShapeDtypeStruct(((_P - _SKIP) * _B, 3), jnp.float32),
        mesh=mesh,
        compiler_params=pltpu.CompilerParams(needs_layout_passes=False,
                                             use_tc_tiling_on_sc=True),
        scratch_types=[
            pltpu.VMEM((_HALF, _P), jnp.int32),
            pltpu.VMEM((224,), jnp.float32),
            pltpu.VMEM((_HALF,), jnp.float32),
            pltpu.VMEM((_HALF,), jnp.float32),
            pltpu.VMEM((_HALF, 3), jnp.float32),
            pltpu.SemaphoreType.DMA,
        ],
    )
    return run(index, pt_t, tab)


# speedup vs baseline: 24.5691x; 1.6015x over previous
import jax
import jax.numpy as jnp
from jax import lax
from jax.experimental import pallas as pl
from jax.experimental.pallas import tpu as pltpu
from jax.experimental.pallas import tpu_sc as plsc

_B = 16384
_P = 200
_NF = 32
_SKIP = 10
_NC = 2
_NS = 16
_NW = _NC * _NS
_L = 16
_CHUNK = _B // _NW
_HALF = _CHUNK // 2
_G = _HALF // _L
_NROWS = (_P - _SKIP) * _B
_NBLK = _NROWS // 256


def _ifs_body(idx_hbm, pt_hbm, tab_hbm, out_hbm, idx_v, tab_v, x_v, y_v,
              stage_v, sem):
    wid = lax.axis_index("s") * _NC + lax.axis_index("c")
    base = wid * _CHUNK

    pltpu.sync_copy(tab_hbm, tab_v)
    lanes = lax.iota(jnp.int32, _L)

    for h in range(2):
        hb = base + h * _HALF
        pltpu.sync_copy(idx_hbm.at[pl.ds(hb, _HALF), :], idx_v)
        pltpu.sync_copy(pt_hbm.at[0, pl.ds(hb, _HALF)], x_v)
        pltpu.sync_copy(pt_hbm.at[1, pl.ds(hb, _HALF)], y_v)

        def step(p, carry):
            pcol = jnp.full((_L,), p, jnp.int32)

            for g in range(_G):
                rows = g * _L + lanes
                fidx = plsc.load_gather(idx_v, [rows, pcol])
                w00 = plsc.load_gather(tab_v, [fidx])
                w01 = plsc.load_gather(tab_v, [fidx + _NF])
                w10 = plsc.load_gather(tab_v, [fidx + 2 * _NF])
                w11 = plsc.load_gather(tab_v, [fidx + 3 * _NF])
                b0 = plsc.load_gather(tab_v, [fidx + 4 * _NF])
                b1 = plsc.load_gather(tab_v, [fidx + 5 * _NF])
                op = plsc.load_gather(tab_v, [fidx + 6 * _NF])
                x = x_v[pl.ds(g * _L, _L)]
                y = y_v[pl.ds(g * _L, _L)]
                nx = w00 * x + w01 * y + b0
                ny = w10 * x + w11 * y + b1
                x_v[pl.ds(g * _L, _L)] = nx
                y_v[pl.ds(g * _L, _L)] = ny
                sub = 4 * (g // 8)
                col = (g % 8) * _L
                stage_v[sub + 0, pl.ds(col, _L)] = nx
                stage_v[sub + 1, pl.ds(col, _L)] = ny
                stage_v[sub + 2, pl.ds(col, _L)] = op

            @pl.when(p >= _SKIP)
            def _():
                blk = ((p - _SKIP) * _B + hb) // 256
                pltpu.async_copy(stage_v, out_hbm.at[blk], sem).wait()

            return carry

        lax.fori_loop(0, _P, step, 0)


def kernel(point, optimized_weights, optimized_biases, optimized_function_ops,
           code):
    logits = jnp.log(code + 1e-8)
    index = jax.random.categorical(jax.random.key(123), logits[:, None, :],
                                   axis=-1, shape=(_B, _P)).astype(jnp.int32)

    tab = jnp.concatenate([
        optimized_weights[:, 0, 0], optimized_weights[:, 0, 1],
        optimized_weights[:, 1, 0], optimized_weights[:, 1, 1],
        optimized_biases[:, 0, 0], optimized_biases[:, 1, 0],
        optimized_function_ops,
    ])
    pt_t = point[:, :, 0].T

    mesh = plsc.VectorSubcoreMesh(core_axis_name="c", subcore_axis_name="s")
    run = pl.kernel(
        _ifs_body,
        out_type=jax.ShapeDtypeStruct((_NBLK, 8, 128), jnp.float32),
        mesh=mesh,
        compiler_params=pltpu.CompilerParams(needs_layout_passes=False,
                                             use_tc_tiling_on_sc=True),
        scratch_types=[
            pltpu.VMEM((_HALF, _P), jnp.int32),
            pltpu.VMEM((224,), jnp.float32),
            pltpu.VMEM((_HALF,), jnp.float32),
            pltpu.VMEM((_HALF,), jnp.float32),
            pltpu.VMEM((8, 128), jnp.float32),
            pltpu.SemaphoreType.DMA,
        ],
    )
    out = run(index, pt_t, tab)
    return (out.reshape(2 * _NBLK, 4, 128)
               .transpose(0, 2, 1)[:, :, :3]
               .reshape(_NROWS, 3))
